# baseline (speedup 1.0000x reference)
import functools

import jax
import jax.numpy as jnp
from jax import lax
from jax.experimental import pallas as pl
from jax.experimental.pallas import tpu as pltpu
from jax.experimental.pallas import tpu_sc as plsc

N_NODES = 10000
N_EDGES = 320000
D = 128

NC = 2
NS = 16
NW = NC * NS
CHUNK = 128
CHUNKS_PER_W = 80
HALF = CHUNKS_PER_W // 2
EDGES_PER_W = CHUNK * CHUNKS_PER_W
E_PAD = NW * EDGES_PER_W
ROWS_PER_TILE = 640
ACC_ROWS = NS * ROWS_PER_TILE


def _seg_body(h_hbm, srcp_hbm, dstp_hbm, out_hbm,
              src_v, dst_v, rows0, rows1, acc, sem0, sem1, ssem0, ssem1):
    cid = lax.axis_index("c")
    sid = lax.axis_index("s")
    wid = cid * NS + sid

    zvec = jnp.zeros((16,), jnp.float32)

    def zrow(r, _):
        for c in range(8):
            rows0[r, pl.ds(c * 16, 16)] = zvec
        return 0

    lax.fori_loop(0, CHUNK, zrow, 0)
    for i in range(ROWS_PER_TILE // CHUNK):
        pltpu.sync_copy(rows0, acc.at[pl.ds(sid * ROWS_PER_TILE + i * CHUNK, CHUNK)])

    for half in range(2):
        pltpu.sync_copy(srcp_hbm.at[wid, pl.ds(half * HALF, HALF)], src_v)
        pltpu.sync_copy(dstp_hbm.at[wid, pl.ds(half * HALF, HALF)], dst_v)

        pltpu.async_copy(h_hbm.at[src_v.at[0]], rows0, sem0)
        pltpu.async_copy(h_hbm.at[src_v.at[1]], rows1, sem1)

        if half == 0:
            plsc.subcore_barrier()

        def body(g, _):
            c0 = 2 * g

            pltpu.make_async_copy(h_hbm.at[src_v.at[c0]], rows0, sem0).wait()
            pltpu.async_copy(rows0, acc.at[dst_v.at[c0]], ssem0, add=True)

            pltpu.make_async_copy(h_hbm.at[src_v.at[c0 + 1]], rows1, sem1).wait()
            pltpu.async_copy(rows1, acc.at[dst_v.at[c0 + 1]], ssem1, add=True)

            pltpu.make_async_copy(rows0, acc.at[dst_v.at[c0]], ssem0).wait()

            @pl.when(c0 + 2 < HALF)
            def _():
                pltpu.async_copy(h_hbm.at[src_v.at[c0 + 2]], rows0, sem0)

            pltpu.make_async_copy(rows1, acc.at[dst_v.at[c0 + 1]], ssem1).wait()

            @pl.when(c0 + 3 < HALF)
            def _():
                pltpu.async_copy(h_hbm.at[src_v.at[c0 + 3]], rows1, sem1)

            return 0

        lax.fori_loop(0, HALF // 2, body, 0)

    plsc.subcore_barrier()
    pltpu.sync_copy(acc.at[pl.ds(sid * ROWS_PER_TILE, ROWS_PER_TILE)],
                    out_hbm.at[cid, pl.ds(sid * ROWS_PER_TILE, ROWS_PER_TILE)])


_seg_sum = pl.kernel(
    _seg_body,
    out_type=jax.ShapeDtypeStruct((NC, ACC_ROWS, D), jnp.float32),
    mesh=plsc.VectorSubcoreMesh(core_axis_name="c", subcore_axis_name="s",
                                num_cores=NC, num_subcores=NS),
    scratch_types=[
        pltpu.VMEM((HALF, CHUNK), jnp.int32),
        pltpu.VMEM((HALF, CHUNK), jnp.int32),
        pltpu.VMEM((CHUNK, D), jnp.float32),
        pltpu.VMEM((CHUNK, D), jnp.float32),
        pltpu.VMEM_SHARED((ACC_ROWS, D), jnp.float32),
        pltpu.SemaphoreType.DMA,
        pltpu.SemaphoreType.DMA,
        pltpu.SemaphoreType.DMA,
        pltpu.SemaphoreType.DMA,
    ],
)


ROW_BLK = 400
GRID = N_NODES // ROW_BLK


def _mm_body(x_ref, w_ref, o_ref):
    o_ref[...] = lax.dot_general(x_ref[...], w_ref[...],
                                 (((1,), (1,)), ((), ())),
                                 preferred_element_type=jnp.float32)


def _matmul_wt(x, w):
    return pl.pallas_call(
        _mm_body,
        grid=(GRID,),
        in_specs=[
            pl.BlockSpec((ROW_BLK, D), lambda i: (i, 0)),
            pl.BlockSpec((D, D), lambda i: (0, 0)),
        ],
        out_specs=pl.BlockSpec((ROW_BLK, D), lambda i: (i, 0)),
        out_shape=jax.ShapeDtypeStruct((N_NODES, D), jnp.float32),
    )(x, w)


def _fuse_mm_body(p0_ref, p1_ref, w_ref, o_ref):
    h = jax.nn.relu(p0_ref[0] + p1_ref[0])
    o_ref[...] = lax.dot_general(h, w_ref[...], (((1,), (1,)), ((), ())),
                                 preferred_element_type=jnp.float32)


def _fused_matmul_wt(p, w):
    return pl.pallas_call(
        _fuse_mm_body,
        grid=(GRID,),
        in_specs=[
            pl.BlockSpec((1, ROW_BLK, D), lambda i: (0, i, 0)),
            pl.BlockSpec((1, ROW_BLK, D), lambda i: (1, i, 0)),
            pl.BlockSpec((D, D), lambda i: (0, 0)),
        ],
        out_specs=pl.BlockSpec((ROW_BLK, D), lambda i: (i, 0)),
        out_shape=jax.ShapeDtypeStruct((N_NODES, D), jnp.float32),
    )(p, p, w)


def _relu_sum_body(p0_ref, p1_ref, o_ref):
    o_ref[...] = jax.nn.relu(p0_ref[0] + p1_ref[0])


def _relu_sum(p):
    return pl.pallas_call(
        _relu_sum_body,
        grid=(GRID,),
        in_specs=[
            pl.BlockSpec((1, ROW_BLK, D), lambda i: (0, i, 0)),
            pl.BlockSpec((1, ROW_BLK, D), lambda i: (1, i, 0)),
        ],
        out_specs=pl.BlockSpec((ROW_BLK, D), lambda i: (i, 0)),
        out_shape=jax.ShapeDtypeStruct((N_NODES, D), jnp.float32),
    )(p, p)


def kernel(X_mask, edge_index, W1, W2):
    dst = edge_index[0].astype(jnp.int32)
    src = edge_index[1].astype(jnp.int32)
    pad = E_PAD - N_EDGES
    pad_src = jnp.arange(pad, dtype=jnp.int32) % N_NODES
    pad_dst = N_NODES + jnp.arange(pad, dtype=jnp.int32) % (ACC_ROWS - N_NODES)
    srcp = jnp.concatenate([src, pad_src]).reshape(NW, CHUNKS_PER_W, CHUNK)
    dstp = jnp.concatenate([dst, pad_dst]).reshape(NW, CHUNKS_PER_W, CHUNK)

    h = _matmul_wt(X_mask, W1)
    p = _seg_sum(h, srcp, dstp)
    h2 = _fused_matmul_wt(p, W2)
    q = _seg_sum(h2, srcp, dstp)
    return _relu_sum(q)


# speedup vs baseline: 1.2461x; 1.2461x over previous
import functools

import jax
import jax.numpy as jnp
from jax import lax
from jax.experimental import pallas as pl
from jax.experimental.pallas import tpu as pltpu
from jax.experimental.pallas import tpu_sc as plsc

N_NODES = 10000
N_EDGES = 320000
D = 128

NC = 2
NS = 16
NW = NC * NS
CHUNK = 128
CHUNKS_PER_W = 80
HALF = CHUNKS_PER_W // 2
EDGES_PER_W = CHUNK * CHUNKS_PER_W
E_PAD = NW * EDGES_PER_W
ROWS_PER_TILE = 640
ACC_ROWS = NS * ROWS_PER_TILE


def _seg_body(h_hbm, srcp_hbm, dstp_hbm, out_hbm,
              src_v, dst_v, rows0, rows1, acc, sem0, sem1, ssem0, ssem1):
    cid = lax.axis_index("c")
    sid = lax.axis_index("s")
    wid = cid * NS + sid

    zvec = jnp.zeros((16,), jnp.float32)

    def zrow(r, _):
        for c in range(8):
            rows0[r, pl.ds(c * 16, 16)] = zvec
        return 0

    lax.fori_loop(0, CHUNK, zrow, 0)
    for i in range(ROWS_PER_TILE // CHUNK):
        pltpu.sync_copy(rows0, acc.at[pl.ds(sid * ROWS_PER_TILE + i * CHUNK, CHUNK)])

    for half in range(2):
        pltpu.sync_copy(srcp_hbm.at[wid, pl.ds(half * HALF, HALF)], src_v)
        pltpu.sync_copy(dstp_hbm.at[wid, pl.ds(half * HALF, HALF)], dst_v)

        pltpu.async_copy(h_hbm.at[src_v.at[0]], rows0, sem0)
        pltpu.async_copy(h_hbm.at[src_v.at[1]], rows1, sem1)

        if half == 0:
            plsc.subcore_barrier()

        def body(g, _):
            c0 = 2 * g

            pltpu.make_async_copy(h_hbm.at[src_v.at[c0]], rows0, sem0).wait()
            pltpu.sync_copy(rows0, acc.at[dst_v.at[c0]], add=True)

            @pl.when(c0 + 2 < HALF)
            def _():
                pltpu.async_copy(h_hbm.at[src_v.at[c0 + 2]], rows0, sem0)

            pltpu.make_async_copy(h_hbm.at[src_v.at[c0 + 1]], rows1, sem1).wait()
            pltpu.sync_copy(rows1, acc.at[dst_v.at[c0 + 1]], add=True)

            @pl.when(c0 + 3 < HALF)
            def _():
                pltpu.async_copy(h_hbm.at[src_v.at[c0 + 3]], rows1, sem1)

            return 0

        lax.fori_loop(0, HALF // 2, body, 0)

    plsc.subcore_barrier()
    pltpu.sync_copy(acc.at[pl.ds(sid * ROWS_PER_TILE, ROWS_PER_TILE)],
                    out_hbm.at[cid, pl.ds(sid * ROWS_PER_TILE, ROWS_PER_TILE)])


_seg_sum = pl.kernel(
    _seg_body,
    out_type=jax.ShapeDtypeStruct((NC, ACC_ROWS, D), jnp.float32),
    mesh=plsc.VectorSubcoreMesh(core_axis_name="c", subcore_axis_name="s",
                                num_cores=NC, num_subcores=NS),
    scratch_types=[
        pltpu.VMEM((HALF, CHUNK), jnp.int32),
        pltpu.VMEM((HALF, CHUNK), jnp.int32),
        pltpu.VMEM((CHUNK, D), jnp.float32),
        pltpu.VMEM((CHUNK, D), jnp.float32),
        pltpu.VMEM_SHARED((ACC_ROWS, D), jnp.float32),
        pltpu.SemaphoreType.DMA,
        pltpu.SemaphoreType.DMA,
        pltpu.SemaphoreType.DMA,
        pltpu.SemaphoreType.DMA,
    ],
)


ROW_BLK = 400
GRID = N_NODES // ROW_BLK


def _mm_body(x_ref, w_ref, o_ref):
    o_ref[...] = lax.dot_general(x_ref[...], w_ref[...],
                                 (((1,), (1,)), ((), ())),
                                 preferred_element_type=jnp.float32)


def _matmul_wt(x, w):
    return pl.pallas_call(
        _mm_body,
        grid=(GRID,),
        in_specs=[
            pl.BlockSpec((ROW_BLK, D), lambda i: (i, 0)),
            pl.BlockSpec((D, D), lambda i: (0, 0)),
        ],
        out_specs=pl.BlockSpec((ROW_BLK, D), lambda i: (i, 0)),
        out_shape=jax.ShapeDtypeStruct((N_NODES, D), jnp.float32),
    )(x, w)


def _fuse_mm_body(p0_ref, p1_ref, w_ref, o_ref):
    h = jax.nn.relu(p0_ref[0] + p1_ref[0])
    o_ref[...] = lax.dot_general(h, w_ref[...], (((1,), (1,)), ((), ())),
                                 preferred_element_type=jnp.float32)


def _fused_matmul_wt(p, w):
    return pl.pallas_call(
        _fuse_mm_body,
        grid=(GRID,),
        in_specs=[
            pl.BlockSpec((1, ROW_BLK, D), lambda i: (0, i, 0)),
            pl.BlockSpec((1, ROW_BLK, D), lambda i: (1, i, 0)),
            pl.BlockSpec((D, D), lambda i: (0, 0)),
        ],
        out_specs=pl.BlockSpec((ROW_BLK, D), lambda i: (i, 0)),
        out_shape=jax.ShapeDtypeStruct((N_NODES, D), jnp.float32),
    )(p, p, w)


def _relu_sum_body(p0_ref, p1_ref, o_ref):
    o_ref[...] = jax.nn.relu(p0_ref[0] + p1_ref[0])


def _relu_sum(p):
    return pl.pallas_call(
        _relu_sum_body,
        grid=(GRID,),
        in_specs=[
            pl.BlockSpec((1, ROW_BLK, D), lambda i: (0, i, 0)),
            pl.BlockSpec((1, ROW_BLK, D), lambda i: (1, i, 0)),
        ],
        out_specs=pl.BlockSpec((ROW_BLK, D), lambda i: (i, 0)),
        out_shape=jax.ShapeDtypeStruct((N_NODES, D), jnp.float32),
    )(p, p)


def kernel(X_mask, edge_index, W1, W2):
    dst = edge_index[0].astype(jnp.int32)
    src = edge_index[1].astype(jnp.int32)
    pad = E_PAD - N_EDGES
    pad_src = jnp.arange(pad, dtype=jnp.int32) % N_NODES
    pad_dst = N_NODES + jnp.arange(pad, dtype=jnp.int32) % (ACC_ROWS - N_NODES)
    srcp = jnp.concatenate([src, pad_src]).reshape(NW, CHUNKS_PER_W, CHUNK)
    dstp = jnp.concatenate([dst, pad_dst]).reshape(NW, CHUNKS_PER_W, CHUNK)

    h = _matmul_wt(X_mask, W1)
    p = _seg_sum(h, srcp, dstp)
    h2 = _fused_matmul_wt(p, W2)
    q = _seg_sum(h2, srcp, dstp)
    return _relu_sum(q)


# speedup vs baseline: 1.3151x; 1.0554x over previous
import functools

import jax
import jax.numpy as jnp
from jax import lax
from jax.experimental import pallas as pl
from jax.experimental.pallas import tpu as pltpu
from jax.experimental.pallas import tpu_sc as plsc

N_NODES = 10000
N_EDGES = 320000
D = 128

NC = 2
NS = 16
NW = NC * NS
CHUNK = 128
CHUNKS_PER_W = 80
HALF = CHUNKS_PER_W // 2
EDGES_PER_W = CHUNK * CHUNKS_PER_W
E_PAD = NW * EDGES_PER_W
ROWS_PER_TILE = 640
ACC_ROWS = NS * ROWS_PER_TILE


def _seg_body(h_hbm, srcp_hbm, dstp_hbm, out_hbm,
              src_v, dst_v, rows0, rows1, acc, sem0, sem1, ssem0, ssem1):
    cid = lax.axis_index("c")
    sid = lax.axis_index("s")
    wid = cid * NS + sid

    zvec = jnp.zeros((16,), jnp.float32)

    def zrow(r, _):
        for c in range(8):
            rows0[r, pl.ds(c * 16, 16)] = zvec
        return 0

    lax.fori_loop(0, CHUNK, zrow, 0)
    for i in range(ROWS_PER_TILE // CHUNK):
        pltpu.sync_copy(rows0, acc.at[pl.ds(sid * ROWS_PER_TILE + i * CHUNK, CHUNK)])

    for half in range(2):
        pltpu.sync_copy(srcp_hbm.at[wid, pl.ds(half * HALF, HALF)], src_v)
        pltpu.sync_copy(dstp_hbm.at[wid, pl.ds(half * HALF, HALF)], dst_v)

        pltpu.async_copy(h_hbm.at[src_v.at[0]], rows0, sem0)
        pltpu.async_copy(h_hbm.at[src_v.at[1]], rows1, sem1)

        if half == 0:
            plsc.subcore_barrier()

        def body(g, _):
            c0 = 2 * g

            pltpu.make_async_copy(h_hbm.at[src_v.at[c0]], rows0, sem0).wait()
            pltpu.sync_copy(rows0, acc.at[dst_v.at[c0]], add=True)

            @pl.when(c0 + 2 < HALF)
            def _():
                pltpu.async_copy(h_hbm.at[src_v.at[c0 + 2]], rows0, sem0)

            pltpu.make_async_copy(h_hbm.at[src_v.at[c0 + 1]], rows1, sem1).wait()
            pltpu.sync_copy(rows1, acc.at[dst_v.at[c0 + 1]], add=True)

            @pl.when(c0 + 3 < HALF)
            def _():
                pltpu.async_copy(h_hbm.at[src_v.at[c0 + 3]], rows1, sem1)

            return 0

        lax.fori_loop(0, HALF // 2, body, 0)

    plsc.subcore_barrier()
    pltpu.sync_copy(acc.at[pl.ds(sid * ROWS_PER_TILE, ROWS_PER_TILE)],
                    out_hbm.at[cid, pl.ds(sid * ROWS_PER_TILE, ROWS_PER_TILE)])


_seg_sum = pl.kernel(
    _seg_body,
    out_type=jax.ShapeDtypeStruct((NC, ACC_ROWS, D), jnp.float32),
    mesh=plsc.VectorSubcoreMesh(core_axis_name="c", subcore_axis_name="s",
                                num_cores=NC, num_subcores=NS),
    scratch_types=[
        pltpu.VMEM((HALF, CHUNK), jnp.int32),
        pltpu.VMEM((HALF, CHUNK), jnp.int32),
        pltpu.VMEM((CHUNK, D), jnp.float32),
        pltpu.VMEM((CHUNK, D), jnp.float32),
        pltpu.VMEM_SHARED((ACC_ROWS, D), jnp.float32),
        pltpu.SemaphoreType.DMA,
        pltpu.SemaphoreType.DMA,
        pltpu.SemaphoreType.DMA,
        pltpu.SemaphoreType.DMA,
    ],
)


ROW_BLK = 400
GRID = N_NODES // ROW_BLK


def _mid_body(p0_ref, p1_ref, w1_ref, o_ref):
    o_ref[...] = jax.nn.relu(
        lax.dot_general(p0_ref[0] + p1_ref[0], w1_ref[...],
                        (((1,), (1,)), ((), ())),
                        preferred_element_type=jnp.float32))


def _mid_matmul(p, w1):
    return pl.pallas_call(
        _mid_body,
        grid=(GRID,),
        in_specs=[
            pl.BlockSpec((1, ROW_BLK, D), lambda i: (0, i, 0)),
            pl.BlockSpec((1, ROW_BLK, D), lambda i: (1, i, 0)),
            pl.BlockSpec((D, D), lambda i: (0, 0)),
        ],
        out_specs=pl.BlockSpec((ROW_BLK, D), lambda i: (i, 0)),
        out_shape=jax.ShapeDtypeStruct((N_NODES, D), jnp.float32),
    )(p, p, w1)


def _final_body(q0_ref, q1_ref, w2_ref, o_ref):
    o_ref[...] = jax.nn.relu(
        lax.dot_general(q0_ref[0] + q1_ref[0], w2_ref[...],
                        (((1,), (1,)), ((), ())),
                        preferred_element_type=jnp.float32))


def _final_matmul(q, w2):
    return pl.pallas_call(
        _final_body,
        grid=(GRID,),
        in_specs=[
            pl.BlockSpec((1, ROW_BLK, D), lambda i: (0, i, 0)),
            pl.BlockSpec((1, ROW_BLK, D), lambda i: (1, i, 0)),
            pl.BlockSpec((D, D), lambda i: (0, 0)),
        ],
        out_specs=pl.BlockSpec((ROW_BLK, D), lambda i: (i, 0)),
        out_shape=jax.ShapeDtypeStruct((N_NODES, D), jnp.float32),
    )(q, q, w2)


def kernel(X_mask, edge_index, W1, W2):
    dst = edge_index[0].astype(jnp.int32)
    src = edge_index[1].astype(jnp.int32)
    pad = E_PAD - N_EDGES
    pad_src = jnp.arange(pad, dtype=jnp.int32) % N_NODES
    pad_dst = N_NODES + jnp.arange(pad, dtype=jnp.int32) % (ACC_ROWS - N_NODES)
    srcp = jnp.concatenate([src, pad_src]).reshape(NW, CHUNKS_PER_W, CHUNK)
    dstp = jnp.concatenate([dst, pad_dst]).reshape(NW, CHUNKS_PER_W, CHUNK)

    p = _seg_sum(X_mask, srcp, dstp)
    h1 = _mid_matmul(p, W1)
    q = _seg_sum(h1, srcp, dstp)
    return _final_matmul(q, W2)


# speedup vs baseline: 1.3850x; 1.0532x over previous
import functools

import jax
import jax.numpy as jnp
from jax import lax
from jax.experimental import pallas as pl
from jax.experimental.pallas import tpu as pltpu
from jax.experimental.pallas import tpu_sc as plsc

N_NODES = 10000
N_EDGES = 320000
D = 128

NC = 2
NS = 16
NW = NC * NS
CHUNK = 128
CHUNKS_PER_W = 80
HALF = CHUNKS_PER_W // 2
EDGES_PER_W = CHUNK * CHUNKS_PER_W
E_PAD = NW * EDGES_PER_W
ROWS_PER_TILE = 640
ACC_ROWS = NS * ROWS_PER_TILE


def _seg_body(h_hbm, srcp_hbm, dstp_hbm, out_hbm,
              src_v, dst_v, rows0, rows1, acc, sem0, sem1, ssem0, ssem1):
    cid = lax.axis_index("c")
    sid = lax.axis_index("s")
    wid = cid * NS + sid

    zvec = jnp.zeros((16,), jnp.float32)

    def zrow(r, _):
        for c in range(8):
            rows0[r, pl.ds(c * 16, 16)] = zvec
        return 0

    lax.fori_loop(0, CHUNK, zrow, 0)
    for i in range(ROWS_PER_TILE // CHUNK):
        pltpu.sync_copy(rows0, acc.at[pl.ds(sid * ROWS_PER_TILE + i * CHUNK, CHUNK)])

    for half in range(2):
        pltpu.sync_copy(srcp_hbm.at[wid, pl.ds(half * HALF, HALF)], src_v)
        pltpu.sync_copy(dstp_hbm.at[wid, pl.ds(half * HALF, HALF)], dst_v)

        pltpu.async_copy(h_hbm.at[src_v.at[0]], rows0, sem0)
        pltpu.async_copy(h_hbm.at[src_v.at[1]], rows1, sem1)

        if half == 0:
            plsc.subcore_barrier()

        def body(g, _):
            c0 = 2 * g

            pltpu.make_async_copy(h_hbm.at[src_v.at[c0]], rows0, sem0).wait()
            pltpu.sync_copy(rows0, acc.at[dst_v.at[c0]], add=True)

            @pl.when(c0 + 2 < HALF)
            def _():
                pltpu.async_copy(h_hbm.at[src_v.at[c0 + 2]], rows0, sem0)

            pltpu.make_async_copy(h_hbm.at[src_v.at[c0 + 1]], rows1, sem1).wait()
            pltpu.sync_copy(rows1, acc.at[dst_v.at[c0 + 1]], add=True)

            @pl.when(c0 + 3 < HALF)
            def _():
                pltpu.async_copy(h_hbm.at[src_v.at[c0 + 3]], rows1, sem1)

            return 0

        lax.fori_loop(0, HALF // 2, body, 0)

    plsc.subcore_barrier()
    pltpu.sync_copy(acc.at[pl.ds(sid * ROWS_PER_TILE, ROWS_PER_TILE)],
                    out_hbm.at[cid, pl.ds(sid * ROWS_PER_TILE, ROWS_PER_TILE)])


_seg_sum = pl.kernel(
    _seg_body,
    out_type=jax.ShapeDtypeStruct((NC, ACC_ROWS, D), jnp.float32),
    mesh=plsc.VectorSubcoreMesh(core_axis_name="c", subcore_axis_name="s",
                                num_cores=NC, num_subcores=NS),
    scratch_types=[
        pltpu.VMEM((HALF, CHUNK), jnp.int32),
        pltpu.VMEM((HALF, CHUNK), jnp.int32),
        pltpu.VMEM((CHUNK, D), jnp.float32),
        pltpu.VMEM((CHUNK, D), jnp.float32),
        pltpu.VMEM_SHARED((ACC_ROWS, D), jnp.float32),
        pltpu.SemaphoreType.DMA,
        pltpu.SemaphoreType.DMA,
        pltpu.SemaphoreType.DMA,
        pltpu.SemaphoreType.DMA,
    ],
)


ROW_BLK = 1000
GRID = N_NODES // ROW_BLK


def _mid_body(p0_ref, p1_ref, w1_ref, o_ref):
    o_ref[...] = jax.nn.relu(
        lax.dot_general(p0_ref[0] + p1_ref[0], w1_ref[...],
                        (((1,), (1,)), ((), ())),
                        preferred_element_type=jnp.float32))


def _mid_matmul(p, w1):
    return pl.pallas_call(
        _mid_body,
        grid=(GRID,),
        in_specs=[
            pl.BlockSpec((1, ROW_BLK, D), lambda i: (0, i, 0)),
            pl.BlockSpec((1, ROW_BLK, D), lambda i: (1, i, 0)),
            pl.BlockSpec((D, D), lambda i: (0, 0)),
        ],
        out_specs=pl.BlockSpec((ROW_BLK, D), lambda i: (i, 0)),
        out_shape=jax.ShapeDtypeStruct((N_NODES, D), jnp.float32),
    )(p, p, w1)


def _final_body(q0_ref, q1_ref, w2_ref, o_ref):
    o_ref[...] = jax.nn.relu(
        lax.dot_general(q0_ref[0] + q1_ref[0], w2_ref[...],
                        (((1,), (1,)), ((), ())),
                        preferred_element_type=jnp.float32))


def _final_matmul(q, w2):
    return pl.pallas_call(
        _final_body,
        grid=(GRID,),
        in_specs=[
            pl.BlockSpec((1, ROW_BLK, D), lambda i: (0, i, 0)),
            pl.BlockSpec((1, ROW_BLK, D), lambda i: (1, i, 0)),
            pl.BlockSpec((D, D), lambda i: (0, 0)),
        ],
        out_specs=pl.BlockSpec((ROW_BLK, D), lambda i: (i, 0)),
        out_shape=jax.ShapeDtypeStruct((N_NODES, D), jnp.float32),
    )(q, q, w2)


def kernel(X_mask, edge_index, W1, W2):
    dst = edge_index[0].astype(jnp.int32)
    src = edge_index[1].astype(jnp.int32)
    pad = E_PAD - N_EDGES
    pad_src = jnp.arange(pad, dtype=jnp.int32) % N_NODES
    pad_dst = N_NODES + jnp.arange(pad, dtype=jnp.int32) % (ACC_ROWS - N_NODES)
    srcp = jnp.concatenate([src, pad_src]).reshape(NW, CHUNKS_PER_W, CHUNK)
    dstp = jnp.concatenate([dst, pad_dst]).reshape(NW, CHUNKS_PER_W, CHUNK)

    p = _seg_sum(X_mask, srcp, dstp)
    h1 = _mid_matmul(p, W1)
    q = _seg_sum(h1, srcp, dstp)
    return _final_matmul(q, W2)


# speedup vs baseline: 1.4170x; 1.0231x over previous
import functools

import jax
import jax.numpy as jnp
from jax import lax
from jax.experimental import pallas as pl
from jax.experimental.pallas import tpu as pltpu
from jax.experimental.pallas import tpu_sc as plsc

N_NODES = 10000
N_EDGES = 320000
D = 128

NC = 2
NS = 16
NW = NC * NS
CHUNK = 128
CHUNKS_PER_W = 80
HALF = CHUNKS_PER_W // 2
EDGES_PER_W = CHUNK * CHUNKS_PER_W
E_PAD = NW * EDGES_PER_W
ROWS_PER_TILE = 640
ACC_ROWS = NS * ROWS_PER_TILE


def _seg_body(h_hbm, srcp_hbm, dstp_hbm, out_hbm,
              src_v, dst_v, rows0, rows1, acc, sem0, sem1, ssem0, ssem1):
    cid = lax.axis_index("c")
    sid = lax.axis_index("s")
    wid = cid * NS + sid

    zvec = jnp.zeros((16,), jnp.float32)

    def zrow(r, _):
        for c in range(8):
            rows0[r, pl.ds(c * 16, 16)] = zvec
        return 0

    lax.fori_loop(0, CHUNK, zrow, 0)
    for i in range(ROWS_PER_TILE // CHUNK):
        pltpu.sync_copy(rows0, acc.at[pl.ds(sid * ROWS_PER_TILE + i * CHUNK, CHUNK)])

    for half in range(2):
        pltpu.sync_copy(srcp_hbm.at[wid, pl.ds(half * HALF, HALF)], src_v)
        pltpu.sync_copy(dstp_hbm.at[wid, pl.ds(half * HALF, HALF)], dst_v)

        pltpu.async_copy(h_hbm.at[src_v.at[0]], rows0, sem0)
        pltpu.async_copy(h_hbm.at[src_v.at[1]], rows1, sem1)

        if half == 0:
            plsc.subcore_barrier()

        def body(g, _):
            c0 = 2 * g

            pltpu.make_async_copy(h_hbm.at[src_v.at[c0]], rows0, sem0).wait()
            pltpu.sync_copy(rows0, acc.at[dst_v.at[c0]], add=True)

            @pl.when(c0 + 2 < HALF)
            def _():
                pltpu.async_copy(h_hbm.at[src_v.at[c0 + 2]], rows0, sem0)

            pltpu.make_async_copy(h_hbm.at[src_v.at[c0 + 1]], rows1, sem1).wait()
            pltpu.sync_copy(rows1, acc.at[dst_v.at[c0 + 1]], add=True)

            @pl.when(c0 + 3 < HALF)
            def _():
                pltpu.async_copy(h_hbm.at[src_v.at[c0 + 3]], rows1, sem1)

            return 0

        lax.fori_loop(0, HALF // 2, body, 0)

    plsc.subcore_barrier()
    pltpu.sync_copy(acc.at[pl.ds(sid * ROWS_PER_TILE, ROWS_PER_TILE)],
                    out_hbm.at[cid, pl.ds(sid * ROWS_PER_TILE, ROWS_PER_TILE)])


_seg_sum = pl.kernel(
    _seg_body,
    out_type=jax.ShapeDtypeStruct((NC, ACC_ROWS, D), jnp.float32),
    mesh=plsc.VectorSubcoreMesh(core_axis_name="c", subcore_axis_name="s",
                                num_cores=NC, num_subcores=NS),
    scratch_types=[
        pltpu.VMEM((HALF, CHUNK), jnp.int32),
        pltpu.VMEM((HALF, CHUNK), jnp.int32),
        pltpu.VMEM((CHUNK, D), jnp.float32),
        pltpu.VMEM((CHUNK, D), jnp.float32),
        pltpu.VMEM_SHARED((ACC_ROWS, D), jnp.float32),
        pltpu.SemaphoreType.DMA,
        pltpu.SemaphoreType.DMA,
        pltpu.SemaphoreType.DMA,
        pltpu.SemaphoreType.DMA,
    ],
)


ROW_BLK = 2000
GRID = N_NODES // ROW_BLK


def _mid_body(p0_ref, p1_ref, w1_ref, o_ref):
    o_ref[...] = jax.nn.relu(
        lax.dot_general(p0_ref[0] + p1_ref[0], w1_ref[...],
                        (((1,), (1,)), ((), ())),
                        preferred_element_type=jnp.float32))


def _mid_matmul(p, w1):
    return pl.pallas_call(
        _mid_body,
        grid=(GRID,),
        in_specs=[
            pl.BlockSpec((1, ROW_BLK, D), lambda i: (0, i, 0)),
            pl.BlockSpec((1, ROW_BLK, D), lambda i: (1, i, 0)),
            pl.BlockSpec((D, D), lambda i: (0, 0)),
        ],
        out_specs=pl.BlockSpec((ROW_BLK, D), lambda i: (i, 0)),
        out_shape=jax.ShapeDtypeStruct((N_NODES, D), jnp.float32),
    )(p, p, w1)


def _final_body(q0_ref, q1_ref, w2_ref, o_ref):
    o_ref[...] = jax.nn.relu(
        lax.dot_general(q0_ref[0] + q1_ref[0], w2_ref[...],
                        (((1,), (1,)), ((), ())),
                        preferred_element_type=jnp.float32))


def _final_matmul(q, w2):
    return pl.pallas_call(
        _final_body,
        grid=(GRID,),
        in_specs=[
            pl.BlockSpec((1, ROW_BLK, D), lambda i: (0, i, 0)),
            pl.BlockSpec((1, ROW_BLK, D), lambda i: (1, i, 0)),
            pl.BlockSpec((D, D), lambda i: (0, 0)),
        ],
        out_specs=pl.BlockSpec((ROW_BLK, D), lambda i: (i, 0)),
        out_shape=jax.ShapeDtypeStruct((N_NODES, D), jnp.float32),
    )(q, q, w2)


def kernel(X_mask, edge_index, W1, W2):
    dst = edge_index[0].astype(jnp.int32)
    src = edge_index[1].astype(jnp.int32)
    pad = E_PAD - N_EDGES
    pad_src = jnp.arange(pad, dtype=jnp.int32) % N_NODES
    pad_dst = N_NODES + jnp.arange(pad, dtype=jnp.int32) % (ACC_ROWS - N_NODES)
    srcp = jnp.concatenate([src, pad_src]).reshape(NW, CHUNKS_PER_W, CHUNK)
    dstp = jnp.concatenate([dst, pad_dst]).reshape(NW, CHUNKS_PER_W, CHUNK)

    p = _seg_sum(X_mask, srcp, dstp)
    h1 = _mid_matmul(p, W1)
    q = _seg_sum(h1, srcp, dstp)
    return _final_matmul(q, W2)
